# baseline (speedup 1.0000x reference)
import math

import jax
import jax.numpy as jnp
from jax import lax
from jax.experimental import pallas as pl
from jax.experimental.pallas import tpu as pltpu
from jax.experimental.pallas import tpu_sc as plsc

N = 10000
E = 320000
HIDDEN = 128
HEADS = 8
HEAD_DIM = 16
CH = 4
W = CH * HEAD_DIM
C = 128
NCHUNK = E // C
NSUB = 16
NJ = -(-NCHUNK // NSUB)
NP = 10240
ROWS_PER_TILE = NP // 16


def _qkv_body(h_ref, wq_ref, bq_ref, wk_ref, bk_ref, wv_ref, bv_ref,
              q_ref, k_ref, v_ref):
    hb = h_ref[...]
    q_ref[...] = jnp.dot(hb, wq_ref[...],
                         preferred_element_type=jnp.float32) + bq_ref[...]
    k_ref[...] = jnp.dot(hb, wk_ref[...],
                         preferred_element_type=jnp.float32) + bk_ref[...]
    v_ref[...] = jnp.dot(hb, wv_ref[...],
                         preferred_element_type=jnp.float32) + bv_ref[...]


def _qkv(h, Wq, bq, Wk, bk, Wv, bv):
    blk = 400
    grid = N // blk
    row_spec = pl.BlockSpec((blk, HIDDEN), lambda i: (i, 0))
    w_spec = pl.BlockSpec((HIDDEN, HIDDEN), lambda i: (0, 0))
    b_spec = pl.BlockSpec((1, HIDDEN), lambda i: (0, 0))
    out = jax.ShapeDtypeStruct((N, HIDDEN), jnp.float32)
    return pl.pallas_call(
        _qkv_body,
        grid=(grid,),
        in_specs=[row_spec, w_spec, b_spec, w_spec, b_spec, w_spec, b_spec],
        out_specs=[row_spec, row_spec, row_spec],
        out_shape=[out, out, out],
    )(h, Wq, bq.reshape(1, HIDDEN), Wk, bk.reshape(1, HIDDEN),
      Wv, bv.reshape(1, HIDDEN))


def _sc_body(q_hbm, k_hbm, v_hbm, x0_hbm, x1_hbm, x2_hbm,
             src_hbm, dst_hbm, dist_hbm, wd_hbm, bd_hbm,
             u_out, a_out,
             u_sh, a_sh, qb, kb, vb, ar,
             xd0, xd1, xd2, srcb, dstb, distb,
             wdb, bdb, semg):
    c = lax.axis_index("c")
    s = lax.axis_index("s")
    qt = q_hbm.at[c]
    kt = k_hbm.at[c]
    vt = v_hbm.at[c]

    zv = jnp.zeros((16,), jnp.float32)

    def zero_row(i, carry):
        for jj in range(W // 16):
            vb[0][i, pl.ds(jj * 16, 16)] = zv
        ar[0][i, pl.ds(0, 16)] = zv
        return carry

    lax.fori_loop(0, C, zero_row, 0)
    for t in range(ROWS_PER_TILE // C):
        rs = pl.ds(s * ROWS_PER_TILE + t * C, C)
        pltpu.sync_copy(vb[0], u_sh.at[rs])
        pltpu.sync_copy(ar[0], a_sh.at[rs])
    pltpu.sync_copy(wd_hbm.at[c], wdb)
    pltpu.sync_copy(bd_hbm.at[c], bdb)
    plsc.subcore_barrier()

    wdv = wdb[...]
    bdv = bdb[...]
    wds = [wdv[h] for h in range(CH)]
    bds = [bdv[h] for h in range(CH)]
    inv_sqrt = 1.0 / math.sqrt(float(HEAD_DIM))
    il = lax.broadcasted_iota(jnp.int32, (16,), 0)
    quart = il >> 2
    hmask = [(il & 3) == h for h in range(CH)]
    himask = (il & 2) != 0

    def gather_list(p):
        return [(qt.at[srcb[p]], qb[p]), (kt.at[dstb[p]], kb[p]),
                (vt.at[dstb[p]], vb[p]),
                (x0_hbm.at[dstb[p]], xd0[p]), (x1_hbm.at[dstb[p]], xd1[p]),
                (x2_hbm.at[dstb[p]], xd2[p])]

    def issue_gathers(jj, p):
        base = (s + jj * NSUB) * C
        pltpu.sync_copy(src_hbm.at[pl.ds(base, C)], srcb[p])
        pltpu.sync_copy(dst_hbm.at[pl.ds(base, C)], dstb[p])
        pltpu.sync_copy(dist_hbm.at[pl.ds(base, C)], distb[p])
        for src_r, dst_r in gather_list(p):
            pltpu.async_copy(src_r, dst_r, semg[p])

    def wait_gathers(p):
        for src_r, dst_r in gather_list(p):
            pltpu.make_async_copy(src_r, dst_r, semg[p]).wait()

    def compute_chunk(p):
        qbuf, kbuf, vbuf, arow = qb[p], kb[p], vb[p], ar[p]

        def group_body(g, carry2):
            gs = pl.ds(g * 16, 16)
            distv = distb[p][gs]
            d2v = distv * distv
            x0v = xd0[p][gs]
            x1v = xd1[p][gs]
            x2v = xd2[p][gs]
            for l in range(16):
                e = g * 16 + l
                evs = []
                for h in range(CH):
                    hs = pl.ds(h * HEAD_DIM, HEAD_DIM)
                    zh = (jnp.sum(qbuf[e, hs] * kbuf[e, hs]) * inv_sqrt
                          - (d2v[l] * wds[h] + bds[h]))
                    evh = jnp.exp(jnp.full((16,), zh, jnp.float32))
                    vbuf[e, hs] = vbuf[e, hs] * evh
                    evs.append(evh)
                ev01 = jnp.where(hmask[1], evs[1], evs[0])
                ev23 = jnp.where(hmask[3], evs[3], evs[2])
                ev = jnp.where(himask, ev23, ev01)
                sel = jnp.where(
                    quart == 0, 1.0,
                    jnp.where(quart == 1,
                              jnp.full((16,), x0v[l], jnp.float32),
                              jnp.where(quart == 2,
                                        jnp.full((16,), x1v[l], jnp.float32),
                                        jnp.full((16,), x2v[l],
                                                 jnp.float32))))
                arow[e, pl.ds(0, 16)] = ev * sel
            return carry2

        lax.fori_loop(0, C // 16, group_body, 0)
        pltpu.sync_copy(vbuf, u_sh.at[srcb[p]], add=True)
        pltpu.sync_copy(arow, a_sh.at[srcb[p]], add=True)

    @pl.when(s < NCHUNK)
    def _():
        issue_gathers(0, 0)

    def pair_body(j2, carry):
        for p in (0, 1):
            jj = 2 * j2 + p

            @pl.when(s + jj * NSUB < NCHUNK)
            def _():
                @pl.when(s + (jj + 1) * NSUB < NCHUNK)
                def _():
                    issue_gathers(jj + 1, 1 - p)

                wait_gathers(p)
                compute_chunk(p)

        return carry

    lax.fori_loop(0, (NJ + 1) // 2, pair_body, 0)
    plsc.subcore_barrier()
    for t in range(ROWS_PER_TILE // C):
        rs = pl.ds(s * ROWS_PER_TILE + t * C, C)
        pltpu.sync_copy(u_sh.at[rs], vb[0])
        pltpu.sync_copy(vb[0], u_out.at[c, rs])
        pltpu.sync_copy(a_sh.at[rs], ar[0])
        pltpu.sync_copy(ar[0], a_out.at[c, rs])


def _sc_edges(qh, kh, vh, x0, x1, x2, src, dst, distances, wd2, bd2):
    mesh = plsc.VectorSubcoreMesh(core_axis_name="c", subcore_axis_name="s")
    fn = pl.kernel(
        _sc_body,
        out_type=(jax.ShapeDtypeStruct((2, NP, W), jnp.float32),
                  jax.ShapeDtypeStruct((2, NP, 16), jnp.float32)),
        mesh=mesh,
        scratch_types=[
            pltpu.VMEM_SHARED((NP, W), jnp.float32),
            pltpu.VMEM_SHARED((NP, 16), jnp.float32),
            (pltpu.VMEM((C, W), jnp.float32),) * 2,
            (pltpu.VMEM((C, W), jnp.float32),) * 2,
            (pltpu.VMEM((C, W), jnp.float32),) * 2,
            (pltpu.VMEM((C, 16), jnp.float32),) * 2,
            (pltpu.VMEM((C,), jnp.float32),) * 2,
            (pltpu.VMEM((C,), jnp.float32),) * 2,
            (pltpu.VMEM((C,), jnp.float32),) * 2,
            (pltpu.VMEM((C,), jnp.int32),) * 2,
            (pltpu.VMEM((C,), jnp.int32),) * 2,
            (pltpu.VMEM((C,), jnp.float32),) * 2,
            pltpu.VMEM((16,), jnp.float32),
            pltpu.VMEM((16,), jnp.float32),
            (pltpu.SemaphoreType.DMA,) * 2,
        ],
        compiler_params=pltpu.CompilerParams(needs_layout_passes=False,
                                             use_tc_tiling_on_sc=False),
    )
    return fn(qh, kh, vh, x0, x1, x2, src, dst, distances, wd2, bd2)


def _final_body(h_ref, x_ref, u0_ref, u1_ref, a0_ref, a1_ref,
                wo_ref, bo_ref, wg1_ref, bg1_ref, wg2_ref, bg2_ref,
                hout_ref, xout_ref):
    a0 = a0_ref[...]
    a1 = a1_ref[...]
    inv0 = 1.0 / jnp.clip(a0[:, 0:CH], 1e-9, None)
    inv1 = 1.0 / jnp.clip(a1[:, 0:CH], 1e-9, None)
    sel = (lax.broadcasted_iota(jnp.int32, (CH, W), 1) // HEAD_DIM
           == lax.broadcasted_iota(jnp.int32, (CH, W), 0)).astype(jnp.float32)
    hu = jnp.concatenate(
        [u0_ref[...] * jnp.dot(inv0, sel, preferred_element_type=jnp.float32),
         u1_ref[...] * jnp.dot(inv1, sel, preferred_element_type=jnp.float32)],
        axis=1)
    h_out = h_ref[...] + jnp.dot(hu, wo_ref[...],
                                 preferred_element_type=jnp.float32) + bo_ref[...]
    hout_ref[...] = h_out

    m0 = inv0 * (1.0 / HEADS)
    m1 = inv1 * (1.0 / HEADS)
    dx = (jnp.sum(a0[:, 4:8] * m0, axis=1, keepdims=True)
          + jnp.sum(a1[:, 4:8] * m1, axis=1, keepdims=True))
    dy = (jnp.sum(a0[:, 8:12] * m0, axis=1, keepdims=True)
          + jnp.sum(a1[:, 8:12] * m1, axis=1, keepdims=True))
    dz = (jnp.sum(a0[:, 12:16] * m0, axis=1, keepdims=True)
          + jnp.sum(a1[:, 12:16] * m1, axis=1, keepdims=True))
    esum = (jnp.sum(a0[:, 0:4] * m0, axis=1, keepdims=True)
            + jnp.sum(a1[:, 0:4] * m1, axis=1, keepdims=True))

    g1 = jnp.dot(h_out, wg1_ref[...],
                 preferred_element_type=jnp.float32) + bg1_ref[...]
    sl = g1 * jax.nn.sigmoid(g1)
    g2 = jnp.dot(sl, wg2_ref[...],
                 preferred_element_type=jnp.float32) + bg2_ref[...]
    gate = jnp.tanh(g2)

    lane = lax.broadcasted_iota(jnp.int32, (1, 16), 1)
    disp = (dx * (lane == 0) + dy * (lane == 1) + dz * (lane == 2)
            - x_ref[...] * esum)
    xout_ref[...] = x_ref[...] + gate * disp


def _final(h, xp, u0, u1, a0, a1, Wo, bo, Wg1, bg1, Wg2, bg2):
    blk = 400
    grid = N // blk
    row128 = pl.BlockSpec((blk, HIDDEN), lambda i: (i, 0))
    row64 = pl.BlockSpec((blk, W), lambda i: (i, 0))
    row16 = pl.BlockSpec((blk, 16), lambda i: (i, 0))
    w_spec = pl.BlockSpec((HIDDEN, HIDDEN), lambda i: (0, 0))
    b_spec = pl.BlockSpec((1, HIDDEN), lambda i: (0, 0))
    return pl.pallas_call(
        _final_body,
        grid=(grid,),
        in_specs=[row128, row16, row64, row64, row16, row16,
                  w_spec, b_spec, w_spec, b_spec,
                  pl.BlockSpec((HIDDEN, 1), lambda i: (0, 0)),
                  pl.BlockSpec((1, 1), lambda i: (0, 0))],
        out_specs=[row128, row16],
        out_shape=[jax.ShapeDtypeStruct((N, HIDDEN), jnp.float32),
                   jax.ShapeDtypeStruct((N, 16), jnp.float32)],
    )(h, xp, u0, u1, a0, a1, Wo, bo.reshape(1, HIDDEN),
      Wg1, bg1.reshape(1, HIDDEN), Wg2, bg2.reshape(1, 1))


def kernel(h, x, src, dst, distances, Wq, bq, Wk, bk, Wv, bv, Wo, bo,
           Wd, bd, Wg1, bg1, Wg2, bg2):
    q, k, v = _qkv(h, Wq, bq, Wk, bk, Wv, bv)
    qh = jnp.stack([q[:, :W], q[:, W:]])
    kh = jnp.stack([k[:, :W], k[:, W:]])
    vh = jnp.stack([v[:, :W], v[:, W:]])
    wd_flat = Wd.reshape(HEADS)
    wd2 = jnp.stack([jnp.tile(wd_flat[:CH], CH), jnp.tile(wd_flat[CH:], CH)])
    bd2 = jnp.stack([jnp.tile(bd[:CH], CH), jnp.tile(bd[CH:], CH)])
    u2, a2 = _sc_edges(qh, kh, vh, jnp.asarray(x[:, 0]), jnp.asarray(x[:, 1]),
                       jnp.asarray(x[:, 2]), src, dst, distances, wd2, bd2)
    xp = jnp.pad(x, ((0, 0), (0, 13)))
    h_out, xp_out = _final(h, xp, u2[0, :N], u2[1, :N], a2[0, :N], a2[1, :N],
                           Wo, bo, Wg1, bg1, Wg2, bg2)
    return (h_out, xp_out[:, :3])


# speedup vs baseline: 1.2867x; 1.2867x over previous
import math

import jax
import jax.numpy as jnp
from jax import lax
from jax.experimental import pallas as pl
from jax.experimental.pallas import tpu as pltpu
from jax.experimental.pallas import tpu_sc as plsc

N = 10000
E = 320000
HIDDEN = 128
HEADS = 8
HEAD_DIM = 16
CH = 4
W = CH * HEAD_DIM
C = 128
NCHUNK = E // C
NSUB = 16
NJ = -(-NCHUNK // NSUB)
NP = 10240
ROWS_PER_TILE = NP // 16


def _qkv_body(h_ref, wq_ref, bq_ref, wk_ref, bk_ref, wv_ref, bv_ref,
              q_ref, k_ref, v_ref):
    hb = h_ref[...]
    q_ref[...] = jnp.dot(hb, wq_ref[...],
                         preferred_element_type=jnp.float32) + bq_ref[...]
    k_ref[...] = jnp.dot(hb, wk_ref[...],
                         preferred_element_type=jnp.float32) + bk_ref[...]
    v_ref[...] = jnp.dot(hb, wv_ref[...],
                         preferred_element_type=jnp.float32) + bv_ref[...]


def _qkv(h, Wq, bq, Wk, bk, Wv, bv):
    blk = 400
    grid = N // blk
    row_spec = pl.BlockSpec((blk, HIDDEN), lambda i: (i, 0))
    w_spec = pl.BlockSpec((HIDDEN, HIDDEN), lambda i: (0, 0))
    b_spec = pl.BlockSpec((1, HIDDEN), lambda i: (0, 0))
    out = jax.ShapeDtypeStruct((N, HIDDEN), jnp.float32)
    return pl.pallas_call(
        _qkv_body,
        grid=(grid,),
        in_specs=[row_spec, w_spec, b_spec, w_spec, b_spec, w_spec, b_spec],
        out_specs=[row_spec, row_spec, row_spec],
        out_shape=[out, out, out],
    )(h, Wq, bq.reshape(1, HIDDEN), Wk, bk.reshape(1, HIDDEN),
      Wv, bv.reshape(1, HIDDEN))


def _sc_body(q_hbm, k_hbm, v_hbm, x0_hbm, x1_hbm, x2_hbm,
             src_hbm, dst_hbm, dist_hbm, wd_hbm, bd_hbm,
             u_out, a_out,
             u_sh, a_sh, qb, kb, vb, ar,
             xd0, xd1, xd2, srcb, dstb, distb,
             wdb, bdb, semg):
    c = lax.axis_index("c")
    s = lax.axis_index("s")
    qt = q_hbm.at[c]
    kt = k_hbm.at[c]
    vt = v_hbm.at[c]

    zv = jnp.zeros((16,), jnp.float32)

    def zero_row(i, carry):
        for jj in range(W // 16):
            vb[0][i, pl.ds(jj * 16, 16)] = zv
        ar[0][i, pl.ds(0, 16)] = zv
        return carry

    lax.fori_loop(0, C, zero_row, 0)
    for t in range(ROWS_PER_TILE // C):
        rs = pl.ds(s * ROWS_PER_TILE + t * C, C)
        pltpu.sync_copy(vb[0], u_sh.at[rs])
        pltpu.sync_copy(ar[0], a_sh.at[rs])
    pltpu.sync_copy(wd_hbm.at[c], wdb)
    pltpu.sync_copy(bd_hbm.at[c], bdb)
    plsc.subcore_barrier()

    wdd = wdb[...]
    bdd = bdb[...]
    inv_sqrt = 1.0 / math.sqrt(float(HEAD_DIM))
    il = lax.broadcasted_iota(jnp.int32, (16,), 0)
    quart = il >> 2
    hmask = [(il & 3) == h for h in range(CH)]

    def gather_list(p):
        return [(qt.at[srcb[p]], qb[p]), (kt.at[dstb[p]], kb[p]),
                (vt.at[dstb[p]], vb[p]),
                (x0_hbm.at[dstb[p]], xd0[p]), (x1_hbm.at[dstb[p]], xd1[p]),
                (x2_hbm.at[dstb[p]], xd2[p])]

    def issue_gathers(jj, p):
        base = (s + jj * NSUB) * C
        pltpu.sync_copy(src_hbm.at[pl.ds(base, C)], srcb[p])
        pltpu.sync_copy(dst_hbm.at[pl.ds(base, C)], dstb[p])
        pltpu.sync_copy(dist_hbm.at[pl.ds(base, C)], distb[p])
        for src_r, dst_r in gather_list(p):
            pltpu.async_copy(src_r, dst_r, semg[p])

    def wait_gathers(p):
        for src_r, dst_r in gather_list(p):
            pltpu.make_async_copy(src_r, dst_r, semg[p]).wait()

    def compute_chunk(p):
        qbuf, kbuf, vbuf, arow = qb[p], kb[p], vb[p], ar[p]

        def group_body(g, carry2):
            gs = pl.ds(g * 16, 16)
            distv = distb[p][gs]
            d2v = distv * distv
            dxv = xd0[p][gs]
            dyv = xd1[p][gs]
            dzv = xd2[p][gs]
            for l in range(16):
                e = g * 16 + l
                sums = []
                for h in range(CH):
                    hs = pl.ds(h * HEAD_DIM, HEAD_DIM)
                    sums.append(jnp.sum(qbuf[e, hs] * kbuf[e, hs]))
                z = jnp.full((16,), sums[0], jnp.float32)
                for h in range(1, CH):
                    z = jnp.where(hmask[h],
                                  jnp.full((16,), sums[h], jnp.float32), z)
                z = z * inv_sqrt - (jnp.full((16,), d2v[l], jnp.float32)
                                    * wdd + bdd)
                ev = jnp.exp(z)
                sel = jnp.where(
                    quart == 0, 1.0,
                    jnp.where(quart == 1,
                              jnp.full((16,), dxv[l], jnp.float32),
                              jnp.where(quart == 2,
                                        jnp.full((16,), dyv[l], jnp.float32),
                                        jnp.full((16,), dzv[l],
                                                 jnp.float32))))
                arow[e, pl.ds(0, 16)] = ev * sel
                for h in range(CH):
                    hs = pl.ds(h * HEAD_DIM, HEAD_DIM)
                    vbuf[e, hs] = vbuf[e, hs] * jnp.full(
                        (16,), ev[h], jnp.float32)
            return carry2

        lax.fori_loop(0, C // 16, group_body, 0)
        pltpu.sync_copy(vbuf, u_sh.at[srcb[p]], add=True)
        pltpu.sync_copy(arow, a_sh.at[srcb[p]], add=True)

    @pl.when(s < NCHUNK)
    def _():
        issue_gathers(0, 0)

    def pair_body(j2, carry):
        for p in (0, 1):
            jj = 2 * j2 + p

            @pl.when(s + jj * NSUB < NCHUNK)
            def _():
                @pl.when(s + (jj + 1) * NSUB < NCHUNK)
                def _():
                    issue_gathers(jj + 1, 1 - p)

                wait_gathers(p)
                compute_chunk(p)

        return carry

    lax.fori_loop(0, (NJ + 1) // 2, pair_body, 0)
    plsc.subcore_barrier()
    for t in range(ROWS_PER_TILE // C):
        rs = pl.ds(s * ROWS_PER_TILE + t * C, C)
        pltpu.sync_copy(u_sh.at[rs], vb[0])
        pltpu.sync_copy(vb[0], u_out.at[c, rs])
        pltpu.sync_copy(a_sh.at[rs], ar[0])
        pltpu.sync_copy(ar[0], a_out.at[c, rs])


def _sc_edges(qh, kh, vh, x0, x1, x2, src, dst, distances, wd2, bd2):
    mesh = plsc.VectorSubcoreMesh(core_axis_name="c", subcore_axis_name="s")
    fn = pl.kernel(
        _sc_body,
        out_type=(jax.ShapeDtypeStruct((2, NP, W), jnp.float32),
                  jax.ShapeDtypeStruct((2, NP, 16), jnp.float32)),
        mesh=mesh,
        scratch_types=[
            pltpu.VMEM_SHARED((NP, W), jnp.float32),
            pltpu.VMEM_SHARED((NP, 16), jnp.float32),
            (pltpu.VMEM((C, W), jnp.float32),) * 2,
            (pltpu.VMEM((C, W), jnp.float32),) * 2,
            (pltpu.VMEM((C, W), jnp.float32),) * 2,
            (pltpu.VMEM((C, 16), jnp.float32),) * 2,
            (pltpu.VMEM((C,), jnp.float32),) * 2,
            (pltpu.VMEM((C,), jnp.float32),) * 2,
            (pltpu.VMEM((C,), jnp.float32),) * 2,
            (pltpu.VMEM((C,), jnp.int32),) * 2,
            (pltpu.VMEM((C,), jnp.int32),) * 2,
            (pltpu.VMEM((C,), jnp.float32),) * 2,
            pltpu.VMEM((16,), jnp.float32),
            pltpu.VMEM((16,), jnp.float32),
            (pltpu.SemaphoreType.DMA,) * 2,
        ],
        compiler_params=pltpu.CompilerParams(needs_layout_passes=False,
                                             use_tc_tiling_on_sc=False),
    )
    return fn(qh, kh, vh, x0, x1, x2, src, dst, distances, wd2, bd2)


def _final_body(h_ref, x_ref, u0_ref, u1_ref, a0_ref, a1_ref,
                wo_ref, bo_ref, wg1_ref, bg1_ref, wg2_ref, bg2_ref,
                hout_ref, xout_ref):
    a0 = a0_ref[...]
    a1 = a1_ref[...]
    inv0 = 1.0 / jnp.clip(a0[:, 0:CH], 1e-9, None)
    inv1 = 1.0 / jnp.clip(a1[:, 0:CH], 1e-9, None)
    sel = (lax.broadcasted_iota(jnp.int32, (CH, W), 1) // HEAD_DIM
           == lax.broadcasted_iota(jnp.int32, (CH, W), 0)).astype(jnp.float32)
    hu = jnp.concatenate(
        [u0_ref[...] * jnp.dot(inv0, sel, preferred_element_type=jnp.float32),
         u1_ref[...] * jnp.dot(inv1, sel, preferred_element_type=jnp.float32)],
        axis=1)
    h_out = h_ref[...] + jnp.dot(hu, wo_ref[...],
                                 preferred_element_type=jnp.float32) + bo_ref[...]
    hout_ref[...] = h_out

    m0 = inv0 * (1.0 / HEADS)
    m1 = inv1 * (1.0 / HEADS)
    dx = (jnp.sum(a0[:, 4:8] * m0, axis=1, keepdims=True)
          + jnp.sum(a1[:, 4:8] * m1, axis=1, keepdims=True))
    dy = (jnp.sum(a0[:, 8:12] * m0, axis=1, keepdims=True)
          + jnp.sum(a1[:, 8:12] * m1, axis=1, keepdims=True))
    dz = (jnp.sum(a0[:, 12:16] * m0, axis=1, keepdims=True)
          + jnp.sum(a1[:, 12:16] * m1, axis=1, keepdims=True))
    esum = (jnp.sum(a0[:, 0:4] * m0, axis=1, keepdims=True)
            + jnp.sum(a1[:, 0:4] * m1, axis=1, keepdims=True))

    g1 = jnp.dot(h_out, wg1_ref[...],
                 preferred_element_type=jnp.float32) + bg1_ref[...]
    sl = g1 * jax.nn.sigmoid(g1)
    g2 = jnp.dot(sl, wg2_ref[...],
                 preferred_element_type=jnp.float32) + bg2_ref[...]
    gate = jnp.tanh(g2)

    lane = lax.broadcasted_iota(jnp.int32, (1, 16), 1)
    disp = (dx * (lane == 0) + dy * (lane == 1) + dz * (lane == 2)
            - x_ref[...] * esum)
    xout_ref[...] = x_ref[...] + gate * disp


def _final(h, xp, u0, u1, a0, a1, Wo, bo, Wg1, bg1, Wg2, bg2):
    blk = 400
    grid = N // blk
    row128 = pl.BlockSpec((blk, HIDDEN), lambda i: (i, 0))
    row64 = pl.BlockSpec((blk, W), lambda i: (i, 0))
    row16 = pl.BlockSpec((blk, 16), lambda i: (i, 0))
    w_spec = pl.BlockSpec((HIDDEN, HIDDEN), lambda i: (0, 0))
    b_spec = pl.BlockSpec((1, HIDDEN), lambda i: (0, 0))
    return pl.pallas_call(
        _final_body,
        grid=(grid,),
        in_specs=[row128, row16, row64, row64, row16, row16,
                  w_spec, b_spec, w_spec, b_spec,
                  pl.BlockSpec((HIDDEN, 1), lambda i: (0, 0)),
                  pl.BlockSpec((1, 1), lambda i: (0, 0))],
        out_specs=[row128, row16],
        out_shape=[jax.ShapeDtypeStruct((N, HIDDEN), jnp.float32),
                   jax.ShapeDtypeStruct((N, 16), jnp.float32)],
    )(h, xp, u0, u1, a0, a1, Wo, bo.reshape(1, HIDDEN),
      Wg1, bg1.reshape(1, HIDDEN), Wg2, bg2.reshape(1, 1))


def kernel(h, x, src, dst, distances, Wq, bq, Wk, bk, Wv, bv, Wo, bo,
           Wd, bd, Wg1, bg1, Wg2, bg2):
    q, k, v = _qkv(h, Wq, bq, Wk, bk, Wv, bv)
    qh = jnp.stack([q[:, :W], q[:, W:]])
    kh = jnp.stack([k[:, :W], k[:, W:]])
    vh = jnp.stack([v[:, :W], v[:, W:]])
    wd_flat = Wd.reshape(HEADS)
    wd2 = jnp.stack([jnp.tile(wd_flat[:CH], CH), jnp.tile(wd_flat[CH:], CH)])
    bd2 = jnp.stack([jnp.tile(bd[:CH], CH), jnp.tile(bd[CH:], CH)])
    u2, a2 = _sc_edges(qh, kh, vh, jnp.asarray(x[:, 0]), jnp.asarray(x[:, 1]),
                       jnp.asarray(x[:, 2]), src, dst, distances, wd2, bd2)
    xp = jnp.pad(x, ((0, 0), (0, 13)))
    h_out, xp_out = _final(h, xp, u2[0, :N], u2[1, :N], a2[0, :N], a2[1, :N],
                           Wo, bo, Wg1, bg1, Wg2, bg2)
    return (h_out, xp_out[:, :3])


# speedup vs baseline: 1.3542x; 1.0525x over previous
import math

import jax
import jax.numpy as jnp
from jax import lax
from jax.experimental import pallas as pl
from jax.experimental.pallas import tpu as pltpu
from jax.experimental.pallas import tpu_sc as plsc

N = 10000
E = 320000
HIDDEN = 128
HEADS = 8
HEAD_DIM = 16
CH = 4
W = CH * HEAD_DIM
C = 160
NCHUNK = E // C
NSUB = 16
NJ = -(-NCHUNK // NSUB)
NP = 10240
ROWS_PER_TILE = NP // 16


def _qkv_body(h_ref, wq_ref, bq_ref, wk_ref, bk_ref, wv_ref, bv_ref,
              q_ref, k_ref, v_ref):
    hb = h_ref[...]
    q_ref[...] = jnp.dot(hb, wq_ref[...],
                         preferred_element_type=jnp.float32) + bq_ref[...]
    k_ref[...] = jnp.dot(hb, wk_ref[...],
                         preferred_element_type=jnp.float32) + bk_ref[...]
    v_ref[...] = jnp.dot(hb, wv_ref[...],
                         preferred_element_type=jnp.float32) + bv_ref[...]


def _qkv(h, Wq, bq, Wk, bk, Wv, bv):
    blk = 400
    grid = N // blk
    row_spec = pl.BlockSpec((blk, HIDDEN), lambda i: (i, 0))
    w_spec = pl.BlockSpec((HIDDEN, HIDDEN), lambda i: (0, 0))
    b_spec = pl.BlockSpec((1, HIDDEN), lambda i: (0, 0))
    out = jax.ShapeDtypeStruct((N, HIDDEN), jnp.float32)
    return pl.pallas_call(
        _qkv_body,
        grid=(grid,),
        in_specs=[row_spec, w_spec, b_spec, w_spec, b_spec, w_spec, b_spec],
        out_specs=[row_spec, row_spec, row_spec],
        out_shape=[out, out, out],
    )(h, Wq, bq.reshape(1, HIDDEN), Wk, bk.reshape(1, HIDDEN),
      Wv, bv.reshape(1, HIDDEN))


def _sc_body(q_hbm, k_hbm, v_hbm, x0_hbm, x1_hbm, x2_hbm,
             src_hbm, dst_hbm, dist_hbm, wd_hbm, bd_hbm,
             u_out, a_out,
             u_sh, a_sh, qb, kb, vb, ar,
             xd0, xd1, xd2, srcb, dstb, distb,
             wdb, bdb, semg):
    c = lax.axis_index("c")
    s = lax.axis_index("s")
    qt = q_hbm.at[c]
    kt = k_hbm.at[c]
    vt = v_hbm.at[c]

    zv = jnp.zeros((16,), jnp.float32)

    def zero_row(i, carry):
        for jj in range(W // 16):
            vb[0][i, pl.ds(jj * 16, 16)] = zv
        ar[0][i, pl.ds(0, 16)] = zv
        return carry

    lax.fori_loop(0, C, zero_row, 0)
    for t in range(ROWS_PER_TILE // C):
        rs = pl.ds(s * ROWS_PER_TILE + t * C, C)
        pltpu.sync_copy(vb[0], u_sh.at[rs])
        pltpu.sync_copy(ar[0], a_sh.at[rs])
    pltpu.sync_copy(wd_hbm.at[c], wdb)
    pltpu.sync_copy(bd_hbm.at[c], bdb)
    plsc.subcore_barrier()

    wdd = wdb[...]
    bdd = bdb[...]
    inv_sqrt = 1.0 / math.sqrt(float(HEAD_DIM))
    il = lax.broadcasted_iota(jnp.int32, (16,), 0)
    quart = il >> 2
    hmask = [(il & 3) == h for h in range(CH)]

    def gather_list(p):
        return [(qt.at[srcb[p]], qb[p]), (kt.at[dstb[p]], kb[p]),
                (vt.at[dstb[p]], vb[p]),
                (x0_hbm.at[dstb[p]], xd0[p]), (x1_hbm.at[dstb[p]], xd1[p]),
                (x2_hbm.at[dstb[p]], xd2[p])]

    def issue_gathers(jj, p):
        base = (s + jj * NSUB) * C
        pltpu.sync_copy(src_hbm.at[pl.ds(base, C)], srcb[p])
        pltpu.sync_copy(dst_hbm.at[pl.ds(base, C)], dstb[p])
        pltpu.sync_copy(dist_hbm.at[pl.ds(base, C)], distb[p])
        for src_r, dst_r in gather_list(p):
            pltpu.async_copy(src_r, dst_r, semg[p])

    def wait_gathers(p):
        for src_r, dst_r in gather_list(p):
            pltpu.make_async_copy(src_r, dst_r, semg[p]).wait()

    def compute_chunk(p):
        qbuf, kbuf, vbuf, arow = qb[p], kb[p], vb[p], ar[p]

        def group_body(g, carry2):
            gs = pl.ds(g * 16, 16)
            distv = distb[p][gs]
            d2v = distv * distv
            dxv = xd0[p][gs]
            dyv = xd1[p][gs]
            dzv = xd2[p][gs]
            for l in range(16):
                e = g * 16 + l
                sums = []
                for h in range(CH):
                    hs = pl.ds(h * HEAD_DIM, HEAD_DIM)
                    sums.append(jnp.sum(qbuf[e, hs] * kbuf[e, hs]))
                z = jnp.full((16,), sums[0], jnp.float32)
                for h in range(1, CH):
                    z = jnp.where(hmask[h],
                                  jnp.full((16,), sums[h], jnp.float32), z)
                z = z * inv_sqrt - (jnp.full((16,), d2v[l], jnp.float32)
                                    * wdd + bdd)
                ev = jnp.exp(z)
                sel = jnp.where(
                    quart == 0, 1.0,
                    jnp.where(quart == 1,
                              jnp.full((16,), dxv[l], jnp.float32),
                              jnp.where(quart == 2,
                                        jnp.full((16,), dyv[l], jnp.float32),
                                        jnp.full((16,), dzv[l],
                                                 jnp.float32))))
                arow[e, pl.ds(0, 16)] = ev * sel
                for h in range(CH):
                    hs = pl.ds(h * HEAD_DIM, HEAD_DIM)
                    vbuf[e, hs] = vbuf[e, hs] * jnp.full(
                        (16,), ev[h], jnp.float32)
            return carry2

        lax.fori_loop(0, C // 16, group_body, 0)
        pltpu.sync_copy(vbuf, u_sh.at[srcb[p]], add=True)
        pltpu.sync_copy(arow, a_sh.at[srcb[p]], add=True)

    @pl.when(s < NCHUNK)
    def _():
        issue_gathers(0, 0)

    def pair_body(j2, carry):
        for p in (0, 1):
            jj = 2 * j2 + p

            @pl.when(s + jj * NSUB < NCHUNK)
            def _():
                @pl.when(s + (jj + 1) * NSUB < NCHUNK)
                def _():
                    issue_gathers(jj + 1, 1 - p)

                wait_gathers(p)
                compute_chunk(p)

        return carry

    lax.fori_loop(0, (NJ + 1) // 2, pair_body, 0)
    plsc.subcore_barrier()
    for t in range(ROWS_PER_TILE // C):
        rs = pl.ds(s * ROWS_PER_TILE + t * C, C)
        pltpu.sync_copy(u_sh.at[rs], vb[0])
        pltpu.sync_copy(vb[0], u_out.at[c, rs])
        pltpu.sync_copy(a_sh.at[rs], ar[0])
        pltpu.sync_copy(ar[0], a_out.at[c, rs])


def _sc_edges(qh, kh, vh, x0, x1, x2, src, dst, distances, wd2, bd2):
    mesh = plsc.VectorSubcoreMesh(core_axis_name="c", subcore_axis_name="s")
    fn = pl.kernel(
        _sc_body,
        out_type=(jax.ShapeDtypeStruct((2, NP, W), jnp.float32),
                  jax.ShapeDtypeStruct((2, NP, 16), jnp.float32)),
        mesh=mesh,
        scratch_types=[
            pltpu.VMEM_SHARED((NP, W), jnp.float32),
            pltpu.VMEM_SHARED((NP, 16), jnp.float32),
            (pltpu.VMEM((C, W), jnp.float32),) * 2,
            (pltpu.VMEM((C, W), jnp.float32),) * 2,
            (pltpu.VMEM((C, W), jnp.float32),) * 2,
            (pltpu.VMEM((C, 16), jnp.float32),) * 2,
            (pltpu.VMEM((C,), jnp.float32),) * 2,
            (pltpu.VMEM((C,), jnp.float32),) * 2,
            (pltpu.VMEM((C,), jnp.float32),) * 2,
            (pltpu.VMEM((C,), jnp.int32),) * 2,
            (pltpu.VMEM((C,), jnp.int32),) * 2,
            (pltpu.VMEM((C,), jnp.float32),) * 2,
            pltpu.VMEM((16,), jnp.float32),
            pltpu.VMEM((16,), jnp.float32),
            (pltpu.SemaphoreType.DMA,) * 2,
        ],
        compiler_params=pltpu.CompilerParams(needs_layout_passes=False,
                                             use_tc_tiling_on_sc=False),
    )
    return fn(qh, kh, vh, x0, x1, x2, src, dst, distances, wd2, bd2)


def _final_body(h_ref, x_ref, u0_ref, u1_ref, a0_ref, a1_ref,
                wo_ref, bo_ref, wg1_ref, bg1_ref, wg2_ref, bg2_ref,
                hout_ref, xout_ref):
    a0 = a0_ref[...]
    a1 = a1_ref[...]
    inv0 = 1.0 / jnp.clip(a0[:, 0:CH], 1e-9, None)
    inv1 = 1.0 / jnp.clip(a1[:, 0:CH], 1e-9, None)
    sel = (lax.broadcasted_iota(jnp.int32, (CH, W), 1) // HEAD_DIM
           == lax.broadcasted_iota(jnp.int32, (CH, W), 0)).astype(jnp.float32)
    hu = jnp.concatenate(
        [u0_ref[...] * jnp.dot(inv0, sel, preferred_element_type=jnp.float32),
         u1_ref[...] * jnp.dot(inv1, sel, preferred_element_type=jnp.float32)],
        axis=1)
    h_out = h_ref[...] + jnp.dot(hu, wo_ref[...],
                                 preferred_element_type=jnp.float32) + bo_ref[...]
    hout_ref[...] = h_out

    m0 = inv0 * (1.0 / HEADS)
    m1 = inv1 * (1.0 / HEADS)
    dx = (jnp.sum(a0[:, 4:8] * m0, axis=1, keepdims=True)
          + jnp.sum(a1[:, 4:8] * m1, axis=1, keepdims=True))
    dy = (jnp.sum(a0[:, 8:12] * m0, axis=1, keepdims=True)
          + jnp.sum(a1[:, 8:12] * m1, axis=1, keepdims=True))
    dz = (jnp.sum(a0[:, 12:16] * m0, axis=1, keepdims=True)
          + jnp.sum(a1[:, 12:16] * m1, axis=1, keepdims=True))
    esum = (jnp.sum(a0[:, 0:4] * m0, axis=1, keepdims=True)
            + jnp.sum(a1[:, 0:4] * m1, axis=1, keepdims=True))

    g1 = jnp.dot(h_out, wg1_ref[...],
                 preferred_element_type=jnp.float32) + bg1_ref[...]
    sl = g1 * jax.nn.sigmoid(g1)
    g2 = jnp.dot(sl, wg2_ref[...],
                 preferred_element_type=jnp.float32) + bg2_ref[...]
    gate = jnp.tanh(g2)

    lane = lax.broadcasted_iota(jnp.int32, (1, 16), 1)
    disp = (dx * (lane == 0) + dy * (lane == 1) + dz * (lane == 2)
            - x_ref[...] * esum)
    xout_ref[...] = x_ref[...] + gate * disp


def _final(h, xp, u0, u1, a0, a1, Wo, bo, Wg1, bg1, Wg2, bg2):
    blk = 400
    grid = N // blk
    row128 = pl.BlockSpec((blk, HIDDEN), lambda i: (i, 0))
    row64 = pl.BlockSpec((blk, W), lambda i: (i, 0))
    row16 = pl.BlockSpec((blk, 16), lambda i: (i, 0))
    w_spec = pl.BlockSpec((HIDDEN, HIDDEN), lambda i: (0, 0))
    b_spec = pl.BlockSpec((1, HIDDEN), lambda i: (0, 0))
    return pl.pallas_call(
        _final_body,
        grid=(grid,),
        in_specs=[row128, row16, row64, row64, row16, row16,
                  w_spec, b_spec, w_spec, b_spec,
                  pl.BlockSpec((HIDDEN, 1), lambda i: (0, 0)),
                  pl.BlockSpec((1, 1), lambda i: (0, 0))],
        out_specs=[row128, row16],
        out_shape=[jax.ShapeDtypeStruct((N, HIDDEN), jnp.float32),
                   jax.ShapeDtypeStruct((N, 16), jnp.float32)],
    )(h, xp, u0, u1, a0, a1, Wo, bo.reshape(1, HIDDEN),
      Wg1, bg1.reshape(1, HIDDEN), Wg2, bg2.reshape(1, 1))


def kernel(h, x, src, dst, distances, Wq, bq, Wk, bk, Wv, bv, Wo, bo,
           Wd, bd, Wg1, bg1, Wg2, bg2):
    q, k, v = _qkv(h, Wq, bq, Wk, bk, Wv, bv)
    qh = jnp.stack([q[:, :W], q[:, W:]])
    kh = jnp.stack([k[:, :W], k[:, W:]])
    vh = jnp.stack([v[:, :W], v[:, W:]])
    wd_flat = Wd.reshape(HEADS)
    wd2 = jnp.stack([jnp.tile(wd_flat[:CH], CH), jnp.tile(wd_flat[CH:], CH)])
    bd2 = jnp.stack([jnp.tile(bd[:CH], CH), jnp.tile(bd[CH:], CH)])
    u2, a2 = _sc_edges(qh, kh, vh, jnp.asarray(x[:, 0]), jnp.asarray(x[:, 1]),
                       jnp.asarray(x[:, 2]), src, dst, distances, wd2, bd2)
    xp = jnp.pad(x, ((0, 0), (0, 13)))
    h_out, xp_out = _final(h, xp, u2[0, :N], u2[1, :N], a2[0, :N], a2[1, :N],
                           Wo, bo, Wg1, bg1, Wg2, bg2)
    return (h_out, xp_out[:, :3])
